# baseline (speedup 1.0000x reference)
import functools

import jax
import jax.numpy as jnp
from jax import lax
from jax.experimental import pallas as pl
from jax.experimental.pallas import tpu as pltpu
from jax.experimental.pallas import tpu_sc as plsc

B = 16384
D = 128
V = 8
NC = 2
NS = 16
NW = NC * NS
B_PER_W = B // NW
CHUNK = 128
N_CHUNKS = B_PER_W // CHUNK
N_REP = 64
L = 16


def _body(names_hbm, emb_hbm, out_hbm, idx_v, rows_v, sem_g, sem_w):
    wid = lax.axis_index("s") * NC + lax.axis_index("c")
    base = wid * B_PER_W
    pltpu.sync_copy(names_hbm.at[pl.ds(base, B_PER_W)], idx_v)
    iota = lax.iota(jnp.int32, L)
    for v in range(B_PER_W // L):
        off = iota * V + (V * L) * (v % (N_REP // L))
        idx_v[pl.ds(v * L, L)] = idx_v[pl.ds(v * L, L)] + off
    for j in range(N_CHUNKS):
        pltpu.async_copy(
            emb_hbm.at[idx_v.at[pl.ds(j * CHUNK, CHUNK)]],
            rows_v.at[pl.ds(j * CHUNK, CHUNK)],
            sem_g,
        )
    for j in range(N_CHUNKS):
        pltpu.make_async_copy(
            emb_hbm.at[idx_v.at[pl.ds(j * CHUNK, CHUNK)]],
            rows_v.at[pl.ds(j * CHUNK, CHUNK)],
            sem_g,
        ).wait()
        pltpu.async_copy(
            rows_v.at[pl.ds(j * CHUNK, CHUNK)],
            out_hbm.at[pl.ds(base + j * CHUNK, CHUNK)],
            sem_w,
        )
    for j in range(N_CHUNKS):
        pltpu.make_async_copy(
            rows_v.at[pl.ds(j * CHUNK, CHUNK)],
            out_hbm.at[pl.ds(base + j * CHUNK, CHUNK)],
            sem_w,
        ).wait()


@jax.jit
def kernel(names, emb):
    mesh = plsc.VectorSubcoreMesh(core_axis_name="c", subcore_axis_name="s")
    f = pl.kernel(
        _body,
        out_type=jax.ShapeDtypeStruct((B, D), jnp.float32),
        mesh=mesh,
        scratch_types=[
            pltpu.VMEM((B_PER_W,), jnp.int32),
            pltpu.VMEM((B_PER_W, D), jnp.float32),
            pltpu.SemaphoreType.DMA,
            pltpu.SemaphoreType.DMA,
        ],
    )
    emb_rep = jnp.tile(emb, (N_REP, 1))
    return f(names.astype(jnp.int32), emb_rep)


# speedup vs baseline: 1.2332x; 1.2332x over previous
import functools

import jax
import jax.numpy as jnp
from jax import lax
from jax.experimental import pallas as pl
from jax.experimental.pallas import tpu as pltpu
from jax.experimental.pallas import tpu_sc as plsc

B = 16384
D = 128
V = 8
NC = 2
NS = 16
NW = NC * NS
B_PER_W = B // NW
CHUNK = 128
N_CHUNKS = B_PER_W // CHUNK
N_REP = 64
L = 16


def _body(names_hbm, emb_hbm, out_hbm, table_sh, idx_v, rows_v, sem):
    sid = lax.axis_index("s")
    wid = sid * NC + lax.axis_index("c")
    base = wid * B_PER_W

    @pl.when(sid == 0)
    def _stage():
        pltpu.sync_copy(emb_hbm, table_sh)

    pltpu.sync_copy(names_hbm.at[pl.ds(base, B_PER_W)], idx_v)
    iota = lax.iota(jnp.int32, L)
    for v in range(B_PER_W // L):
        off = iota * V + (V * L) * (v % (N_REP // L))
        idx_v[pl.ds(v * L, L)] = idx_v[pl.ds(v * L, L)] + off
    plsc.subcore_barrier()
    for j in range(N_CHUNKS):
        pltpu.async_copy(
            table_sh.at[idx_v.at[pl.ds(j * CHUNK, CHUNK)]],
            rows_v.at[pl.ds(j * CHUNK, CHUNK)],
            sem,
        )
    for j in range(N_CHUNKS):
        pltpu.make_async_copy(
            table_sh.at[idx_v.at[pl.ds(j * CHUNK, CHUNK)]],
            rows_v.at[pl.ds(j * CHUNK, CHUNK)],
            sem,
        ).wait()
    pltpu.sync_copy(rows_v, out_hbm.at[pl.ds(base, B_PER_W)])


@jax.jit
def kernel(names, emb):
    mesh = plsc.VectorSubcoreMesh(core_axis_name="c", subcore_axis_name="s")
    f = pl.kernel(
        _body,
        out_type=jax.ShapeDtypeStruct((B, D), jnp.float32),
        mesh=mesh,
        scratch_types=[
            pltpu.VMEM_SHARED((N_REP * V, D), jnp.float32),
            pltpu.VMEM((B_PER_W,), jnp.int32),
            pltpu.VMEM((B_PER_W, D), jnp.float32),
            pltpu.SemaphoreType.DMA,
        ],
    )
    emb_rep = jnp.tile(emb, (N_REP, 1))
    return f(names.astype(jnp.int32), emb_rep)


# speedup vs baseline: 1.2635x; 1.0246x over previous
import functools

import jax
import jax.numpy as jnp
from jax import lax
from jax.experimental import pallas as pl
from jax.experimental.pallas import tpu as pltpu
from jax.experimental.pallas import tpu_sc as plsc

B = 16384
D = 128
V = 8
NC = 2
NS = 16
NW = NC * NS
B_PER_W = B // NW
CHUNK = 128
N_CHUNKS = B_PER_W // CHUNK
N_REP = 64
REP_PER_TILE = N_REP // NS
L = 16


def _body(names_hbm, emb_hbm, out_hbm, table_sh, tbl_v, idx_v, rows_v, sem_g, sem_w):
    sid = lax.axis_index("s")
    wid = sid * NC + lax.axis_index("c")
    base = wid * B_PER_W

    pltpu.sync_copy(emb_hbm, tbl_v)
    for k in range(REP_PER_TILE):
        pltpu.sync_copy(tbl_v, table_sh.at[pl.ds((sid * REP_PER_TILE + k) * V, V)])

    pltpu.sync_copy(names_hbm.at[pl.ds(base, B_PER_W)], idx_v)
    iota = lax.iota(jnp.int32, L)
    for v in range(B_PER_W // L):
        off = iota * V + (V * L) * (v % (N_REP // L))
        idx_v[pl.ds(v * L, L)] = idx_v[pl.ds(v * L, L)] + off
    plsc.subcore_barrier()
    for j in range(N_CHUNKS):
        pltpu.async_copy(
            table_sh.at[idx_v.at[pl.ds(j * CHUNK, CHUNK)]],
            rows_v.at[pl.ds(j * CHUNK, CHUNK)],
            sem_g,
        )
    for j in range(N_CHUNKS):
        pltpu.make_async_copy(
            table_sh.at[idx_v.at[pl.ds(j * CHUNK, CHUNK)]],
            rows_v.at[pl.ds(j * CHUNK, CHUNK)],
            sem_g,
        ).wait()
        pltpu.async_copy(
            rows_v.at[pl.ds(j * CHUNK, CHUNK)],
            out_hbm.at[pl.ds(base + j * CHUNK, CHUNK)],
            sem_w,
        )
    for j in range(N_CHUNKS):
        pltpu.make_async_copy(
            rows_v.at[pl.ds(j * CHUNK, CHUNK)],
            out_hbm.at[pl.ds(base + j * CHUNK, CHUNK)],
            sem_w,
        ).wait()


@jax.jit
def kernel(names, emb):
    mesh = plsc.VectorSubcoreMesh(core_axis_name="c", subcore_axis_name="s")
    f = pl.kernel(
        _body,
        out_type=jax.ShapeDtypeStruct((B, D), jnp.float32),
        mesh=mesh,
        scratch_types=[
            pltpu.VMEM_SHARED((N_REP * V, D), jnp.float32),
            pltpu.VMEM((V, D), jnp.float32),
            pltpu.VMEM((B_PER_W,), jnp.int32),
            pltpu.VMEM((B_PER_W, D), jnp.float32),
            pltpu.SemaphoreType.DMA,
            pltpu.SemaphoreType.DMA,
        ],
    )
    return f(names.astype(jnp.int32), emb)
